# baseline (speedup 1.0000x reference)
import functools

import jax
import jax.numpy as jnp
from jax.experimental import pallas as pl
from jax.experimental.pallas import tpu as pltpu

_E = 8
_K = 2
_D = 768
_F = 3072
_BLK = 256
_GQ = 65535.0
_INTERPRET = False


def _router_body(tok_ref, wg_ref, pos_ref, packed_ref, blk_ref, *, nb):
    logits = jnp.dot(tok_ref[...], wg_ref[...], preferred_element_type=jnp.float32)
    t, c = logits.shape
    col = jax.lax.broadcasted_iota(jnp.int32, (t, c), 1)
    lg = jnp.where(col < _E, logits, -1e30)
    m1 = jnp.max(lg, axis=1, keepdims=True)
    i1 = jnp.min(jnp.where(lg == m1, col, c), axis=1, keepdims=True)
    lg2 = jnp.where(col == i1, -1e30, lg)
    m2 = jnp.max(lg2, axis=1, keepdims=True)
    i2 = jnp.min(jnp.where(lg2 == m2, col, c), axis=1, keepdims=True)
    g1 = 1.0 / (1.0 + jnp.exp(m2 - m1))

    row = jax.lax.broadcasted_iota(jnp.int32, (t, c), 0)
    q1 = jnp.round(g1 * _GQ).astype(jnp.int32)
    q2 = jnp.round((1.0 - g1) * _GQ).astype(jnp.int32)
    packed_ref[...] = row * 65536 + jnp.where(col == 0, q1,
                                              jnp.where(col == 1, q2, 0))

    oh2 = ((col == i1) | (col == i2)).astype(jnp.float32)
    counts = jnp.sum(oh2, axis=0, keepdims=True)
    nblk = jnp.floor((counts + (_BLK - 1)) * (1.0 / _BLK))
    lane_r = jax.lax.broadcasted_iota(jnp.int32, (c, c), 0)
    lane_c = jax.lax.broadcasted_iota(jnp.int32, (c, c), 1)
    upper = (lane_r < lane_c).astype(jnp.float32)
    blk_start = jnp.dot(nblk, upper, preferred_element_type=jnp.float32)
    off = blk_start * float(_BLK)

    inc = oh2
    k = 1
    while k < t:
        shifted = jnp.concatenate(
            [jnp.zeros((k, c), jnp.float32), inc[: t - k]], axis=0)
        inc = inc + shifted
        k *= 2
    rank = inc - oh2

    sel1 = (col == i1).astype(jnp.float32)
    sel2 = (col == i2).astype(jnp.float32)
    offb = jnp.broadcast_to(off, (t, c))
    pos1 = jnp.sum(sel1 * (offb + rank), axis=1, keepdims=True)
    pos2 = jnp.sum(sel2 * (offb + rank), axis=1, keepdims=True)
    pos_ref[...] = jnp.where(col == 0, pos1, jnp.where(col == 1, pos2, 0.0)
                             ).astype(jnp.int32)

    incl = (blk_start + nblk).astype(jnp.int32)
    jrow = jax.lax.broadcasted_iota(jnp.int32, (nb, c), 0)
    inclb = jnp.broadcast_to(incl, (nb, c))
    lane2 = jax.lax.broadcasted_iota(jnp.int32, (nb, c), 1)
    hit = ((inclb <= jrow) & (lane2 < _E)).astype(jnp.int32)
    blk_ref[...] = jnp.minimum(jnp.sum(hit, axis=1, keepdims=True), _E - 1
                               ) + jnp.zeros((nb, c), jnp.int32)


def _ffn_body(tbl_ref, blke_ref,
              tok_ref, w1_ref, b1_ref, w2_ref, b2_ref, lng_ref, lnb_ref,
              acc_ref, xb_ref, yb_ref, *, nb, t):
    i = pl.program_id(0)

    @pl.when(i == 0)
    def _():
        acc_ref[...] = tok_ref[...]

    def gather(r, _):
        tk = jax.lax.shift_right_logical(tbl_ref[i * _BLK + r], 16)
        xb_ref[r, :] = tok_ref[tk, :]
        return 0

    jax.lax.fori_loop(0, _BLK, gather, 0, unroll=8)

    h = jnp.dot(xb_ref[...], w1_ref[0], preferred_element_type=jnp.float32)
    h = jnp.maximum(h + b1_ref[0], 0.0)
    y = jnp.dot(h, w2_ref[0], preferred_element_type=jnp.float32) + b2_ref[0]
    yb_ref[...] = y

    def scatter(r, _):
        v = tbl_ref[i * _BLK + r]
        tk = jax.lax.shift_right_logical(v, 16)
        g = (v & 65535).astype(jnp.float32) * (1.0 / _GQ)
        acc_ref[tk, :] = acc_ref[tk, :] + g * yb_ref[r, :]
        return 0

    jax.lax.fori_loop(0, _BLK, scatter, 0, unroll=8)

    @pl.when(i == nb - 1)
    def _():
        a = acc_ref[...]
        m = jnp.mean(a, axis=1, keepdims=True)
        v = jnp.mean((a - m) * (a - m), axis=1, keepdims=True)
        acc_ref[...] = (a - m) * jax.lax.rsqrt(v + 1e-5) * lng_ref[...] + lnb_ref[...]


def kernel(x, Wg, W1, b1, W2, b2, ln_g, ln_b):
    bv, nv, dv = x.shape
    t = bv * nv
    p = t * _K
    nb = (p + _BLK - 1) // _BLK + (_E - 1)
    nr = nb * _BLK

    tokens = x.reshape(t, dv)

    wgp = jnp.zeros((dv, 128), Wg.dtype).at[:, :_E].set(Wg)
    pos_out, packed_out, blk_out = pl.pallas_call(
        functools.partial(_router_body, nb=nb),
        out_shape=(
            jax.ShapeDtypeStruct((t, 128), jnp.int32),
            jax.ShapeDtypeStruct((t, 128), jnp.int32),
            jax.ShapeDtypeStruct((nb, 128), jnp.int32),
        ),
        interpret=_INTERPRET,
    )(tokens, wgp)

    pos2 = pos_out[:, :_K].reshape(p)
    packed = packed_out[:, :_K].reshape(p)
    blk_e = blk_out[:, 0]
    table = jnp.full((nr,), (t - 1) * 65536, jnp.int32).at[pos2].set(packed)

    b1r = b1.reshape(_E, 1, _F)
    b2r = b2.reshape(_E, 1, dv)
    lngr = ln_g.reshape(1, dv)
    lnbr = ln_b.reshape(1, dv)

    grid_spec = pltpu.PrefetchScalarGridSpec(
        num_scalar_prefetch=2,
        grid=(nb,),
        in_specs=[
            pl.BlockSpec((t, dv), lambda i, tb, be: (0, 0)),
            pl.BlockSpec((1, dv, _F), lambda i, tb, be: (be[i], 0, 0)),
            pl.BlockSpec((1, 1, _F), lambda i, tb, be: (be[i], 0, 0)),
            pl.BlockSpec((1, _F, dv), lambda i, tb, be: (be[i], 0, 0)),
            pl.BlockSpec((1, 1, dv), lambda i, tb, be: (be[i], 0, 0)),
            pl.BlockSpec((1, dv), lambda i, tb, be: (0, 0)),
            pl.BlockSpec((1, dv), lambda i, tb, be: (0, 0)),
        ],
        out_specs=pl.BlockSpec((t, dv), lambda i, tb, be: (0, 0)),
        scratch_shapes=[
            pltpu.VMEM((_BLK, dv), jnp.float32),
            pltpu.VMEM((_BLK, dv), jnp.float32),
        ],
    )
    acc = pl.pallas_call(
        functools.partial(_ffn_body, nb=nb, t=t),
        grid_spec=grid_spec,
        out_shape=jax.ShapeDtypeStruct((t, dv), jnp.float32),
        compiler_params=pltpu.CompilerParams(
            dimension_semantics=("arbitrary",)),
        interpret=_INTERPRET,
    )(table, blk_e,
      tokens, W1, b1r, W2, b2r, lngr, lnbr)

    return acc.reshape(bv, nv, dv)
